# baseline (speedup 1.0000x reference)
import functools

import jax
import jax.numpy as jnp
from jax import lax
from jax.experimental import pallas as pl
from jax.experimental.pallas import tpu as pltpu
from jax.experimental.pallas import tpu_sc as plsc

VOCAB = 100000
DIM = 256
H = 256
CLASSES = 10
B = 1024
T = 50
CH = 10
NCH = T // CH


def _sc_gather(table_w, idx):
    info = plsc.get_sparse_core_info()
    nc, ns = info.num_cores, info.num_subcores
    nw = nc * ns
    n = idx.shape[0]
    w = table_w.shape[1]
    per_w = n // nw
    ch = 200
    n_ch = per_w // ch
    mesh = plsc.VectorSubcoreMesh(core_axis_name="c", subcore_axis_name="s")

    @functools.partial(
        pl.kernel,
        mesh=mesh,
        out_type=jax.ShapeDtypeStruct((n, w), jnp.int32),
        scratch_types=[
            pltpu.VMEM((ch,), jnp.int32),
            pltpu.VMEM((ch, w), jnp.int32),
            pltpu.SemaphoreType.DMA,
        ],
    )
    def k(table_hbm, idx_hbm, out_hbm, idx_v, rows_v, sem):
        wid = lax.axis_index("s") * nc + lax.axis_index("c")
        for c_i in range(n_ch):
            base = wid * per_w + c_i * ch
            pltpu.sync_copy(idx_hbm.at[pl.ds(base, ch)], idx_v)
            pltpu.async_copy(table_hbm.at[idx_v], rows_v, sem).wait()
            pltpu.sync_copy(rows_v, out_hbm.at[pl.ds(base, ch)])

    return k(table_w, idx)


def _lstm_gates(gates, c):
    ii = 0.5 * jnp.tanh(gates[:, :H]) + 0.5
    ff = 0.5 * jnp.tanh(gates[:, H:2 * H]) + 0.5
    gg = jnp.tanh(gates[:, 2 * H:3 * H])
    oo = 0.5 * jnp.tanh(gates[:, 3 * H:]) + 0.5
    c2 = ff * c + ii * gg
    h2 = oo * jnp.tanh(c2)
    return h2, c2


def _gate_scale():
    return jnp.concatenate([jnp.full((2 * H,), 0.5, jnp.float32),
                            jnp.ones((H,), jnp.float32),
                            jnp.full((H,), 0.5, jnp.float32)])


def _scan_kernel(reverse, nin, emb_refs, w_refs, whh_ref, b_ref, out_ref,
                 h_ref, c_ref, last_only):
    c_idx = pl.program_id(0)

    @pl.when(c_idx == 0)
    def _():
        h_ref[...] = jnp.zeros_like(h_ref)
        c_ref[...] = jnp.zeros_like(c_ref)

    for j in range(CH):
        jj = CH - 1 - j if reverse else j
        gates = b_ref[...]
        for r, w in zip(emb_refs, w_refs):
            gates = gates + jnp.dot(r[jj], w[...],
                                    preferred_element_type=jnp.float32)
        gates = gates + jnp.dot(h_ref[...], whh_ref[...],
                                preferred_element_type=jnp.float32)
        h2, c2 = _lstm_gates(gates, c_ref[...])
        h2b = h2.astype(jnp.bfloat16)
        h_ref[...] = h2b
        c_ref[...] = c2
        if last_only:
            if j == CH - 1:
                @pl.when(c_idx == NCH - 1)
                def _():
                    out_ref[...] = h2b
        else:
            out_ref[0, jj] = h2b


def _scan(embs, ws, whh, b, reverse, last_only):
    nin = len(embs)
    cmap = (lambda c: (NCH - 1 - c,) + (0,) * 2) if reverse \
        else (lambda c: (c, 0, 0))
    in_specs = [pl.BlockSpec((CH, B, e.shape[-1]), cmap) for e in embs]
    in_specs += [pl.BlockSpec(w.shape, lambda c: (0,) * w.ndim)
                 for w in ws + [whh, b]]
    if last_only:
        out_spec = pl.BlockSpec((B, H), lambda c: (0, 0))
        out_shape = jax.ShapeDtypeStruct((B, H), jnp.bfloat16)
    else:
        out_spec = pl.BlockSpec((1, CH, B, H),
                                lambda c, _m=cmap: (0,) + _m(c))
        out_shape = jax.ShapeDtypeStruct((1, T, B, H), jnp.bfloat16)

    def body(*refs):
        emb_refs = refs[:nin]
        w_refs = refs[nin:2 * nin]
        whh_ref, b_ref, out_ref, h_ref, c_ref = refs[2 * nin:]
        _scan_kernel(reverse, nin, emb_refs, w_refs, whh_ref, b_ref,
                     out_ref, h_ref, c_ref, last_only)

    out = pl.pallas_call(
        body,
        grid=(NCH,),
        in_specs=in_specs,
        out_specs=out_spec,
        out_shape=out_shape,
        scratch_shapes=[
            pltpu.VMEM((B, H), jnp.bfloat16),
            pltpu.VMEM((B, H), jnp.float32),
        ],
    )(*embs, *ws, whh, b)
    return out if last_only else out[0]


def _final_kernel(hfl_ref, hbl_ref, hf1_ref, wa_ref, wb_ref, b_ref,
                  wfa_ref, wfb_ref, bfc_ref, out_ref):
    gates = (
        jnp.dot(hfl_ref[...], wa_ref[...], preferred_element_type=jnp.float32)
        + jnp.dot(hbl_ref[...], wb_ref[...], preferred_element_type=jnp.float32)
        + b_ref[...]
    )
    h2, _ = _lstm_gates(gates, jnp.zeros((B, H), jnp.float32))
    h2 = h2.astype(jnp.bfloat16)
    out_ref[...] = (
        jnp.dot(hf1_ref[...], wfa_ref[...], preferred_element_type=jnp.float32)
        + jnp.dot(h2, wfb_ref[...], preferred_element_type=jnp.float32)
        + bfc_ref[...]
    )


def _final(hf0_last, hb0_last, hf1, wa, wb, b, wfa, wfb, bfc_row):
    npad = wfa.shape[1]
    return pl.pallas_call(
        _final_kernel,
        out_shape=jax.ShapeDtypeStruct((B, npad), jnp.float32),
    )(hf0_last, hb0_last, hf1, wa, wb, b, wfa, wfb, bfc_row)


def kernel(x, table, Wih0f, Whh0f, bih0f, bhh0f, Wih0b, Whh0b, bih0b, bhh0b,
           Wih1f, Whh1f, bih1f, bhh1f, Wih1b, Whh1b, bih1b, bhh1b, Wfc, bfc):
    bf = jnp.bfloat16
    s = _gate_scale()

    idx = x.astype(jnp.int32).T.reshape(-1)
    table_w = lax.bitcast_convert_type(
        table.astype(bf).reshape(VOCAB, DIM // 2, 2), jnp.int32)
    emb = lax.bitcast_convert_type(
        _sc_gather(table_w, idx), bf).reshape(T, B, DIM)

    hf0 = _scan([emb], [(Wih0f.T * s).astype(bf)], (Whh0f.T * s).astype(bf),
                ((bih0f + bhh0f) * s).reshape(1, -1), False, False)
    hb0 = _scan([emb], [(Wih0b.T * s).astype(bf)], (Whh0b.T * s).astype(bf),
                ((bih0b + bhh0b) * s).reshape(1, -1), True, False)

    w1f = (Wih1f.T * s).astype(bf)
    hf1 = _scan([hf0, hb0], [w1f[:H], w1f[H:]], (Whh1f.T * s).astype(bf),
                ((bih1f + bhh1f) * s).reshape(1, -1), False, True)

    w1b = (Wih1b.T * s).astype(bf)
    npad = 128
    wfc_t = jnp.zeros((2 * H, npad), jnp.float32).at[:, :CLASSES].set(Wfc.T)
    wfc_t = wfc_t.astype(bf)
    bfc_row = jnp.zeros((1, npad), jnp.float32).at[:, :CLASSES].set(bfc)
    logits = _final(hf0[T - 1], hb0[T - 1], hf1,
                    w1b[:H], w1b[H:],
                    ((bih1b + bhh1b) * s).reshape(1, -1),
                    wfc_t[:H], wfc_t[H:], bfc_row)
    return logits[:, :CLASSES]


# speedup vs baseline: 3.0862x; 3.0862x over previous
import functools

import jax
import jax.numpy as jnp
from jax import lax
from jax.experimental import pallas as pl
from jax.experimental.pallas import tpu as pltpu
from jax.experimental.pallas import tpu_sc as plsc

VOCAB = 100000
DIM = 256
H = 256
CLASSES = 10
B = 1024
T = 50
CH = 10
NCH = T // CH

_RT = (((1,), (1,)), ((), ()))


def _sc_gather(table, idx):
    info = plsc.get_sparse_core_info()
    nc, ns = info.num_cores, info.num_subcores
    nw = nc * ns
    n = idx.shape[0]
    d = table.shape[1]
    per_w = n // nw
    ch = 200
    n_ch = per_w // ch
    mesh = plsc.VectorSubcoreMesh(core_axis_name="c", subcore_axis_name="s")

    @functools.partial(
        pl.kernel,
        mesh=mesh,
        out_type=jax.ShapeDtypeStruct((n, d), jnp.float32),
        scratch_types=[
            pltpu.VMEM((ch,), jnp.int32),
            pltpu.VMEM((ch, d), jnp.float32),
            pltpu.SemaphoreType.DMA,
        ],
    )
    def k(table_hbm, idx_hbm, out_hbm, idx_v, rows_v, sem):
        wid = lax.axis_index("s") * nc + lax.axis_index("c")
        for c_i in range(n_ch):
            base = wid * per_w + c_i * ch
            pltpu.sync_copy(idx_hbm.at[pl.ds(base, ch)], idx_v)
            pltpu.async_copy(table_hbm.at[idx_v], rows_v, sem).wait()
            pltpu.sync_copy(rows_v, out_hbm.at[pl.ds(base, ch)])

    return k(table, idx)


def _lstm_gates(gates, c):
    ii = 0.5 * jnp.tanh(gates[:, :H]) + 0.5
    ff = 0.5 * jnp.tanh(gates[:, H:2 * H]) + 0.5
    gg = jnp.tanh(gates[:, 2 * H:3 * H])
    oo = 0.5 * jnp.tanh(gates[:, 3 * H:]) + 0.5
    c2 = ff * c + ii * gg
    h2 = oo * jnp.tanh(c2)
    return h2, c2


def _gate_scale():
    return jnp.concatenate([jnp.full((2 * H, 1), 0.5, jnp.float32),
                            jnp.ones((H, 1), jnp.float32),
                            jnp.full((H, 1), 0.5, jnp.float32)])


def _scan_kernel(reverse, nin, emb_refs, w_refs, whh_ref, b_ref, out_ref,
                 h_ref, c_ref, last_only):
    c_idx = pl.program_id(0)

    @pl.when(c_idx == 0)
    def _():
        h_ref[...] = jnp.zeros_like(h_ref)
        c_ref[...] = jnp.zeros_like(c_ref)

    for j in range(CH):
        jj = CH - 1 - j if reverse else j
        gates = b_ref[...]
        for r, w in zip(emb_refs, w_refs):
            gates = gates + lax.dot_general(
                r[jj].astype(jnp.bfloat16), w[...], _RT,
                preferred_element_type=jnp.float32)
        gates = gates + lax.dot_general(
            h_ref[...], whh_ref[...], _RT,
            preferred_element_type=jnp.float32)
        h2, c2 = _lstm_gates(gates, c_ref[...])
        h2b = h2.astype(jnp.bfloat16)
        h_ref[...] = h2b
        c_ref[...] = c2
        if last_only:
            if j == CH - 1:
                @pl.when(c_idx == NCH - 1)
                def _():
                    out_ref[...] = h2b
        else:
            out_ref[jj] = h2b


def _scan(embs, ws, whh, b, reverse, last_only):
    nin = len(embs)
    cmap = (lambda c: (NCH - 1 - c, 0, 0)) if reverse \
        else (lambda c: (c, 0, 0))
    in_specs = [pl.BlockSpec((CH, B, e.shape[-1]), cmap) for e in embs]
    in_specs += [pl.BlockSpec(w.shape, lambda c, _n=w.ndim: (0,) * _n)
                 for w in ws + [whh, b]]
    if last_only:
        out_spec = pl.BlockSpec((B, H), lambda c: (0, 0))
        out_shape = jax.ShapeDtypeStruct((B, H), jnp.bfloat16)
    else:
        out_spec = pl.BlockSpec((CH, B, H), cmap)
        out_shape = jax.ShapeDtypeStruct((T, B, H), jnp.bfloat16)

    def body(*refs):
        emb_refs = refs[:nin]
        w_refs = refs[nin:2 * nin]
        whh_ref, b_ref, out_ref, h_ref, c_ref = refs[2 * nin:]
        _scan_kernel(reverse, nin, emb_refs, w_refs, whh_ref, b_ref,
                     out_ref, h_ref, c_ref, last_only)

    return pl.pallas_call(
        body,
        grid=(NCH,),
        in_specs=in_specs,
        out_specs=out_spec,
        out_shape=out_shape,
        scratch_shapes=[
            pltpu.VMEM((B, H), jnp.bfloat16),
            pltpu.VMEM((B, H), jnp.float32),
        ],
    )(*embs, *ws, whh, b)


def _final_kernel(hfl_ref, hbl_ref, hf1_ref, wa_ref, wb_ref, b_ref,
                  wfa_ref, wfb_ref, bfc_ref, out_ref):
    gates = (
        lax.dot_general(hfl_ref[...], wa_ref[...], _RT,
                        preferred_element_type=jnp.float32)
        + lax.dot_general(hbl_ref[...], wb_ref[...], _RT,
                          preferred_element_type=jnp.float32)
        + b_ref[...]
    )
    h2, _ = _lstm_gates(gates, jnp.zeros((B, H), jnp.float32))
    h2 = h2.astype(jnp.bfloat16)
    out_ref[...] = (
        lax.dot_general(hf1_ref[...], wfa_ref[...], _RT,
                        preferred_element_type=jnp.float32)
        + lax.dot_general(h2, wfb_ref[...], _RT,
                          preferred_element_type=jnp.float32)
        + bfc_ref[...]
    )


def _final(hf0_last, hb0_last, hf1, wa, wb, b, wfa, wfb, bfc_row):
    npad = wfa.shape[0]
    return pl.pallas_call(
        _final_kernel,
        out_shape=jax.ShapeDtypeStruct((B, npad), jnp.float32),
    )(hf0_last, hb0_last, hf1, wa, wb, b, wfa, wfb, bfc_row)


def kernel(x, table, Wih0f, Whh0f, bih0f, bhh0f, Wih0b, Whh0b, bih0b, bhh0b,
           Wih1f, Whh1f, bih1f, bhh1f, Wih1b, Whh1b, bih1b, bhh1b, Wfc, bfc):
    bf = jnp.bfloat16
    s = _gate_scale()
    sv = s.reshape(1, -1)

    idx = x.astype(jnp.int32).T.reshape(-1)
    emb = _sc_gather(table, idx).reshape(T, B, DIM)

    hf0 = _scan([emb], [(Wih0f * s).astype(bf)], (Whh0f * s).astype(bf),
                (bih0f + bhh0f).reshape(1, -1) * sv, False, False)
    hb0 = _scan([emb], [(Wih0b * s).astype(bf)], (Whh0b * s).astype(bf),
                (bih0b + bhh0b).reshape(1, -1) * sv, True, False)

    w1f = (Wih1f * s).astype(bf)
    hf1 = _scan([hf0, hb0], [w1f[:, :H], w1f[:, H:]],
                (Whh1f * s).astype(bf),
                (bih1f + bhh1f).reshape(1, -1) * sv, False, True)

    w1b = (Wih1b * s).astype(bf)
    npad = 128
    wfc_p = jnp.zeros((npad, 2 * H), jnp.float32).at[:CLASSES].set(Wfc)
    wfc_p = wfc_p.astype(bf)
    bfc_row = jnp.zeros((1, npad), jnp.float32).at[:, :CLASSES].set(bfc)
    logits = _final(hf0[T - 1], hb0[T - 1], hf1,
                    w1b[:, :H], w1b[:, H:],
                    (bih1b + bhh1b).reshape(1, -1) * sv,
                    wfc_p[:, :H], wfc_p[:, H:], bfc_row)
    return logits[:, :CLASSES]


# speedup vs baseline: 3.2595x; 1.0561x over previous
import functools

import jax
import jax.numpy as jnp
from jax import lax
from jax.experimental import pallas as pl
from jax.experimental.pallas import tpu as pltpu
from jax.experimental.pallas import tpu_sc as plsc

VOCAB = 100000
DIM = 256
H = 256
CLASSES = 10
B = 1024
T = 50
CH0 = 5
NCH0 = T // CH0
CH1 = 10
NCH1 = T // CH1
NPAD = 128

_RT = (((1,), (1,)), ((), ()))
_F32 = jnp.float32
_BF = jnp.bfloat16


def _sc_gather(table, idx):
    info = plsc.get_sparse_core_info()
    nc, ns = info.num_cores, info.num_subcores
    nw = nc * ns
    n = idx.shape[0]
    d = table.shape[1]
    per_w = n // nw
    ch = 200
    n_ch = per_w // ch
    mesh = plsc.VectorSubcoreMesh(core_axis_name="c", subcore_axis_name="s")

    @functools.partial(
        pl.kernel,
        mesh=mesh,
        out_type=jax.ShapeDtypeStruct((n, d), _F32),
        scratch_types=[
            pltpu.VMEM((ch,), jnp.int32),
            pltpu.VMEM((ch, d), _F32),
            pltpu.SemaphoreType.DMA,
        ],
    )
    def k(table_hbm, idx_hbm, out_hbm, idx_v, rows_v, sem):
        wid = lax.axis_index("s") * nc + lax.axis_index("c")
        for c_i in range(n_ch):
            base = wid * per_w + c_i * ch
            pltpu.sync_copy(idx_hbm.at[pl.ds(base, ch)], idx_v)
            pltpu.async_copy(table_hbm.at[idx_v], rows_v, sem).wait()
            pltpu.sync_copy(rows_v, out_hbm.at[pl.ds(base, ch)])

    return k(table, idx)


def _lstm_gates(gates, c):
    ii = 0.5 * jnp.tanh(gates[:, :H]) + 0.5
    ff = 0.5 * jnp.tanh(gates[:, H:2 * H]) + 0.5
    gg = jnp.tanh(gates[:, 2 * H:3 * H])
    oo = 0.5 * jnp.tanh(gates[:, 3 * H:]) + 0.5
    c2 = ff * c + ii * gg
    h2 = oo * jnp.tanh(c2)
    return h2, c2


def _dot(x, w):
    return lax.dot_general(x, w[...], _RT, preferred_element_type=_F32)


def _l0_kernel(embf_ref, embb_ref, wf_ref, whf_ref, bf_ref,
               wb_ref, whb_ref, bb_ref, outf_ref, outb_ref,
               hf_ref, cf_ref, hb_ref, cb_ref):
    c_idx = pl.program_id(0)

    @pl.when(c_idx == 0)
    def _():
        hf_ref[...] = jnp.zeros_like(hf_ref)
        cf_ref[...] = jnp.zeros_like(cf_ref)
        hb_ref[...] = jnp.zeros_like(hb_ref)
        cb_ref[...] = jnp.zeros_like(cb_ref)

    for j in range(CH0):
        jj = CH0 - 1 - j
        gf = (bf_ref[...] + _dot(embf_ref[j].astype(_BF), wf_ref)
              + _dot(hf_ref[...], whf_ref))
        gb = (bb_ref[...] + _dot(embb_ref[jj].astype(_BF), wb_ref)
              + _dot(hb_ref[...], whb_ref))
        h2f, c2f = _lstm_gates(gf, cf_ref[...])
        h2b, c2b = _lstm_gates(gb, cb_ref[...])
        h2fb = h2f.astype(_BF)
        h2bb = h2b.astype(_BF)
        hf_ref[...] = h2fb
        cf_ref[...] = c2f
        hb_ref[...] = h2bb
        cb_ref[...] = c2b
        outf_ref[j] = h2fb
        outb_ref[jj] = h2bb


def _layer0(emb, wf, whf, bf_, wb, whb, bb_):
    fmap = lambda c: (c, 0, 0)
    rmap = lambda c: (NCH0 - 1 - c, 0, 0)
    wmap2 = lambda c: (0, 0)
    return pl.pallas_call(
        _l0_kernel,
        grid=(NCH0,),
        in_specs=[
            pl.BlockSpec((CH0, B, DIM), fmap),
            pl.BlockSpec((CH0, B, DIM), rmap),
            pl.BlockSpec(wf.shape, wmap2),
            pl.BlockSpec(whf.shape, wmap2),
            pl.BlockSpec(bf_.shape, wmap2),
            pl.BlockSpec(wb.shape, wmap2),
            pl.BlockSpec(whb.shape, wmap2),
            pl.BlockSpec(bb_.shape, wmap2),
        ],
        out_specs=[
            pl.BlockSpec((CH0, B, H), fmap),
            pl.BlockSpec((CH0, B, H), rmap),
        ],
        out_shape=[
            jax.ShapeDtypeStruct((T, B, H), _BF),
            jax.ShapeDtypeStruct((T, B, H), _BF),
        ],
        scratch_shapes=[
            pltpu.VMEM((B, H), _BF),
            pltpu.VMEM((B, H), _F32),
            pltpu.VMEM((B, H), _BF),
            pltpu.VMEM((B, H), _F32),
        ],
    )(emb, emb, wf, whf, bf_, wb, whb, bb_)


def _l1_kernel(hf0_ref, hb0_ref, wa_ref, wb_ref, wh_ref, b_ref,
               wba_ref, wbb_ref, bb_ref, wfa_ref, wfb_ref, bfc_ref,
               out_ref, h_ref, c_ref):
    c_idx = pl.program_id(0)

    @pl.when(c_idx == 0)
    def _():
        h_ref[...] = jnp.zeros_like(h_ref)
        c_ref[...] = jnp.zeros_like(c_ref)

    for j in range(CH1):
        gates = (b_ref[...] + _dot(hf0_ref[j], wa_ref)
                 + _dot(hb0_ref[j], wb_ref) + _dot(h_ref[...], wh_ref))
        h2, c2 = _lstm_gates(gates, c_ref[...])
        h_ref[...] = h2.astype(_BF)
        c_ref[...] = c2
        if j == CH1 - 1:
            @pl.when(c_idx == NCH1 - 1)
            def _():
                gb = (bb_ref[...] + _dot(hf0_ref[j], wba_ref)
                      + _dot(hb0_ref[j], wbb_ref))
                hb1, _ = _lstm_gates(gb, jnp.zeros((B, H), _F32))
                out_ref[...] = (
                    bfc_ref[...] + _dot(h_ref[...], wfa_ref)
                    + _dot(hb1.astype(_BF), wfb_ref))


def _layer1(hf0, hb0, wa, wb, wh, b, wba, wbb, bb_, wfa, wfb, bfc_row):
    fmap = lambda c: (c, 0, 0)
    wmap2 = lambda c: (0, 0)
    consts = [wa, wb, wh, b, wba, wbb, bb_, wfa, wfb, bfc_row]
    return pl.pallas_call(
        _l1_kernel,
        grid=(NCH1,),
        in_specs=[pl.BlockSpec((CH1, B, H), fmap),
                  pl.BlockSpec((CH1, B, H), fmap)]
        + [pl.BlockSpec(w.shape, wmap2) for w in consts],
        out_specs=pl.BlockSpec((B, NPAD), lambda c: (0, 0)),
        out_shape=jax.ShapeDtypeStruct((B, NPAD), _F32),
        scratch_shapes=[
            pltpu.VMEM((B, H), _BF),
            pltpu.VMEM((B, H), _F32),
        ],
    )(hf0, hb0, *consts)


def kernel(x, table, Wih0f, Whh0f, bih0f, bhh0f, Wih0b, Whh0b, bih0b, bhh0b,
           Wih1f, Whh1f, bih1f, bhh1f, Wih1b, Whh1b, bih1b, bhh1b, Wfc, bfc):
    s = jnp.concatenate([jnp.full((2 * H, 1), 0.5, _F32),
                         jnp.ones((H, 1), _F32),
                         jnp.full((H, 1), 0.5, _F32)])
    sv = s.reshape(1, -1)

    idx = x.astype(jnp.int32).T.reshape(-1)
    emb = _sc_gather(table, idx).reshape(T, B, DIM)

    hf0, hb0 = _layer0(
        emb,
        (Wih0f * s).astype(_BF), (Whh0f * s).astype(_BF),
        (bih0f + bhh0f).reshape(1, -1) * sv,
        (Wih0b * s).astype(_BF), (Whh0b * s).astype(_BF),
        (bih0b + bhh0b).reshape(1, -1) * sv,
    )

    w1f = (Wih1f * s).astype(_BF)
    w1b = (Wih1b * s).astype(_BF)
    wfc_p = jnp.zeros((NPAD, 2 * H), _F32).at[:CLASSES].set(Wfc).astype(_BF)
    bfc_row = jnp.zeros((1, NPAD), _F32).at[:, :CLASSES].set(bfc)
    logits = _layer1(
        hf0, hb0,
        w1f[:, :H], w1f[:, H:], (Whh1f * s).astype(_BF),
        (bih1f + bhh1f).reshape(1, -1) * sv,
        w1b[:, :H], w1b[:, H:],
        (bih1b + bhh1b).reshape(1, -1) * sv,
        wfc_p[:, :H], wfc_p[:, H:], bfc_row,
    )
    return logits[:, :CLASSES]
